# Initial kernel scaffold; baseline (speedup 1.0000x reference)
import jax
import jax.numpy as jnp
from jax.experimental import pallas as pl


def kernel(x, indices):
    raise NotImplementedError("write your pallas kernel here")


# speedup vs baseline: 89.8355x; 89.8355x over previous
import functools

import jax
import jax.numpy as jnp
from jax import lax
from jax.experimental import pallas as pl
from jax.experimental.pallas import tpu as pltpu
from jax.experimental.pallas import tpu_sc as plsc

_B, _C, _Hp, _Wp = 4, 96, 192, 192
_H, _W = 384, 384
_N = _B * _C
_PW = _Hp * _Wp
_OW = _H * _W

_NC, _NS, _L = 2, 16, 16
_NW = _NC * _NS
_PPW = _N // _NW

_CH = 32
_IN_CHUNK = _CH * _Wp
_OUT_CHUNK = 2 * _CH * _W
_CHUNKS = _Hp // _CH
_T = _PPW * _CHUNKS
_IN_IT = _IN_CHUNK // _L
_OUT_IT = _OUT_CHUNK // _L
_UNROLL = 8


def _unpool_body(x_hbm, idx_hbm, out_hbm,
                 xv0, xv1, iv0, iv1, rv0, rv1, ov0, ov1,
                 sin0, sin1, sout0, sout1):
    wid = lax.axis_index("s") * _NC + lax.axis_index("c")
    xv = (xv0, xv1)
    iv = (iv0, iv1)
    rv = (rv0, rv1)
    ov = (ov0, ov1)
    sin = (sin0, sin1)
    sout = (sout0, sout1)

    zeros16 = jnp.zeros((_L,), jnp.float32)

    def bases(t):
        plane = wid * _PPW + t // _CHUNKS
        chunk = t % _CHUNKS
        return plane * _PW + chunk * _IN_CHUNK, plane * _OW + chunk * _OUT_CHUNK, chunk * _OUT_CHUNK

    def start_in(t, b):
        in_base, _, _ = bases(t)
        pltpu.async_copy(x_hbm.at[pl.ds(in_base, _IN_CHUNK)], xv[b], sin[b])
        pltpu.async_copy(idx_hbm.at[pl.ds(in_base, _IN_CHUNK)], iv[b], sin[b])

    def wait_in(b):
        pltpu.make_async_copy(x_hbm.at[pl.ds(0, _IN_CHUNK)], xv[b], sin[b]).wait()
        pltpu.make_async_copy(idx_hbm.at[pl.ds(0, _IN_CHUNK)], iv[b], sin[b]).wait()

    def wait_out(b):
        pltpu.make_async_copy(ov[b], out_hbm.at[pl.ds(0, _OUT_CHUNK)], sout[b]).wait()

    def zinit(i, c):
        ov0[pl.ds(i * _L, _L)] = zeros16
        ov1[pl.ds(i * _L, _L)] = zeros16
        return c
    lax.fori_loop(0, _OUT_IT, zinit, 0, unroll=4)

    start_in(0, 0)

    def step_pair(i, c):
        for b in range(2):
            t = 2 * i + b

            @pl.when(i >= 1)
            def _():
                wait_out(b)

                def restore(k, cc):
                    off = rv[b][pl.ds(k * _L, _L)]
                    plsc.store_scatter(ov[b], [off], zeros16)
                    return cc
                lax.fori_loop(0, _IN_IT, restore, 0, unroll=_UNROLL)

            wait_in(b)

            if b == 0:
                start_in(t + 1, 1)
            else:
                @pl.when(i < _T // 2 - 1)
                def _():
                    start_in(t + 1, 0)

            _, out_base, local_base = bases(t)

            def scatter(k, cc):
                xvec = xv[b][pl.ds(k * _L, _L)]
                ivec = iv[b][pl.ds(k * _L, _L)]
                off = ivec - local_base
                rv[b][pl.ds(k * _L, _L)] = off
                plsc.store_scatter(ov[b], [off], xvec)
                return cc
            lax.fori_loop(0, _IN_IT, scatter, 0, unroll=_UNROLL)

            pltpu.async_copy(ov[b], out_hbm.at[pl.ds(out_base, _OUT_CHUNK)], sout[b])
        return c

    lax.fori_loop(0, _T // 2, step_pair, 0)

    wait_out(0)
    wait_out(1)


@functools.partial(jax.jit)
def _unpool(x_flat, idx_flat):
    mesh = plsc.VectorSubcoreMesh(core_axis_name="c", subcore_axis_name="s")
    run = functools.partial(
        pl.kernel,
        mesh=mesh,
        out_type=jax.ShapeDtypeStruct((_N * _OW,), jnp.float32),
        compiler_params=pltpu.CompilerParams(needs_layout_passes=False),
        scratch_types=[
            pltpu.VMEM((_IN_CHUNK,), jnp.float32),
            pltpu.VMEM((_IN_CHUNK,), jnp.float32),
            pltpu.VMEM((_IN_CHUNK,), jnp.int32),
            pltpu.VMEM((_IN_CHUNK,), jnp.int32),
            pltpu.VMEM((_IN_CHUNK,), jnp.int32),
            pltpu.VMEM((_IN_CHUNK,), jnp.int32),
            pltpu.VMEM((_OUT_CHUNK,), jnp.float32),
            pltpu.VMEM((_OUT_CHUNK,), jnp.float32),
            pltpu.SemaphoreType.DMA,
            pltpu.SemaphoreType.DMA,
            pltpu.SemaphoreType.DMA,
            pltpu.SemaphoreType.DMA,
        ],
    )(_unpool_body)
    return run(x_flat, idx_flat)


def kernel(x, indices):
    x_flat = x.reshape(_N * _PW)
    idx_flat = indices.reshape(_N * _PW)
    out = _unpool(x_flat, idx_flat)
    return out.reshape(_B, _C, _H, _W)


# speedup vs baseline: 127.4927x; 1.4192x over previous
import functools

import jax
import jax.numpy as jnp
from jax import lax
from jax.experimental import pallas as pl
from jax.experimental.pallas import tpu as pltpu
from jax.experimental.pallas import tpu_sc as plsc

_B, _C, _Hp, _Wp = 4, 96, 192, 192
_H, _W = 384, 384
_N = _B * _C
_PW = _Hp * _Wp
_OW = _H * _W

_NC, _NS, _L = 2, 16, 16
_NW = _NC * _NS
_PPW = _N // _NW

_CH = 32
_IN_CHUNK = _CH * _Wp
_OUT_CHUNK = 2 * _CH * _W
_CHUNKS = _Hp // _CH
_T = _PPW * _CHUNKS
_IN_IT = _IN_CHUNK // _L
_OUT_IT = _OUT_CHUNK // _L
_UNROLL = 8


def _unpool_body(x_hbm, idx_hbm, out_hbm,
                 xv0, xv1, iv0, iv1, rv0, rv1, ov0, ov1,
                 sin0, sin1, sout0, sout1):
    wid = lax.axis_index("s") * _NC + lax.axis_index("c")
    xv = (xv0, xv1)
    iv = (iv0, iv1)
    rv = (rv0, rv1)
    ov = (ov0, ov1)
    sin = (sin0, sin1)
    sout = (sout0, sout1)

    zeros16 = jnp.zeros((_L,), jnp.float32)

    def bases(t):
        plane = wid * _PPW + t // _CHUNKS
        chunk = t % _CHUNKS
        return plane * _PW + chunk * _IN_CHUNK, plane * _OW + chunk * _OUT_CHUNK, chunk * _OUT_CHUNK

    def start_in(t, b):
        in_base, _, _ = bases(t)
        pltpu.async_copy(x_hbm.at[pl.ds(in_base, _IN_CHUNK)], xv[b], sin[b])
        pltpu.async_copy(idx_hbm.at[pl.ds(in_base, _IN_CHUNK)], iv[b], sin[b])

    def wait_in(b):
        pltpu.make_async_copy(x_hbm.at[pl.ds(0, _IN_CHUNK)], xv[b], sin[b]).wait()
        pltpu.make_async_copy(idx_hbm.at[pl.ds(0, _IN_CHUNK)], iv[b], sin[b]).wait()

    def wait_out(b):
        pltpu.make_async_copy(ov[b], out_hbm.at[pl.ds(0, _OUT_CHUNK)], sout[b]).wait()

    @plsc.parallel_loop(0, _OUT_CHUNK, step=_L, unroll=4)
    def _(i):
        ov0[pl.ds(i, _L)] = zeros16
        ov1[pl.ds(i, _L)] = zeros16

    start_in(0, 0)

    def step_pair(i, c):
        for b in range(2):
            t = 2 * i + b

            @pl.when(i >= 1)
            def _():
                wait_out(b)

                @plsc.parallel_loop(0, _IN_CHUNK, step=_L, unroll=_UNROLL)
                def _(k):
                    off = rv[b][pl.ds(k, _L)]
                    plsc.store_scatter(ov[b], [off], zeros16)

            wait_in(b)

            if b == 0:
                start_in(t + 1, 1)
            else:
                @pl.when(i < _T // 2 - 1)
                def _():
                    start_in(t + 1, 0)

            _, out_base, local_base = bases(t)

            @plsc.parallel_loop(0, _IN_CHUNK, step=_L, unroll=_UNROLL)
            def _(k):
                xvec = xv[b][pl.ds(k, _L)]
                ivec = iv[b][pl.ds(k, _L)]
                off = ivec - local_base
                rv[b][pl.ds(k, _L)] = off
                plsc.store_scatter(ov[b], [off], xvec)

            pltpu.async_copy(ov[b], out_hbm.at[pl.ds(out_base, _OUT_CHUNK)], sout[b])
        return c

    lax.fori_loop(0, _T // 2, step_pair, 0)

    wait_out(0)
    wait_out(1)


@functools.partial(jax.jit)
def _unpool(x_flat, idx_flat):
    mesh = plsc.VectorSubcoreMesh(core_axis_name="c", subcore_axis_name="s")
    run = functools.partial(
        pl.kernel,
        mesh=mesh,
        out_type=jax.ShapeDtypeStruct((_N * _OW,), jnp.float32),
        compiler_params=pltpu.CompilerParams(needs_layout_passes=False),
        scratch_types=[
            pltpu.VMEM((_IN_CHUNK,), jnp.float32),
            pltpu.VMEM((_IN_CHUNK,), jnp.float32),
            pltpu.VMEM((_IN_CHUNK,), jnp.int32),
            pltpu.VMEM((_IN_CHUNK,), jnp.int32),
            pltpu.VMEM((_IN_CHUNK,), jnp.int32),
            pltpu.VMEM((_IN_CHUNK,), jnp.int32),
            pltpu.VMEM((_OUT_CHUNK,), jnp.float32),
            pltpu.VMEM((_OUT_CHUNK,), jnp.float32),
            pltpu.SemaphoreType.DMA,
            pltpu.SemaphoreType.DMA,
            pltpu.SemaphoreType.DMA,
            pltpu.SemaphoreType.DMA,
        ],
    )(_unpool_body)
    return run(x_flat, idx_flat)


def kernel(x, indices):
    x_flat = x.reshape(_N * _PW)
    idx_flat = indices.reshape(_N * _PW)
    out = _unpool(x_flat, idx_flat)
    return out.reshape(_B, _C, _H, _W)


# speedup vs baseline: 276.0155x; 2.1650x over previous
import functools

import jax
import jax.numpy as jnp
from jax import lax
from jax.experimental import pallas as pl
from jax.experimental.pallas import tpu as pltpu
from jax.experimental.pallas import tpu_sc as plsc

_B, _C, _Hp, _Wp = 4, 96, 192, 192
_H, _W = 384, 384
_N = _B * _C

_NC, _NS, _L = 2, 16, 16
_NW = _NC * _NS
_PPW = _N // _NW

_CH = 32
_VPR = _Wp // _L
_CHUNKS = _Hp // _CH
_T = _PPW * _CHUNKS
_OCH = 2 * _CH


def _unpool_body(x_hbm, idx_hbm, out_hbm,
                 xv0, xv1, iv0, iv1, rv0, rv1, ov0, ov1,
                 sin0, sin1, sout0, sout1):
    wid = lax.axis_index("s") * _NC + lax.axis_index("c")
    xv = (xv0, xv1)
    iv = (iv0, iv1)
    rv = (rv0, rv1)
    ov = (ov0, ov1)
    sin = (sin0, sin1)
    sout = (sout0, sout1)

    zeros16 = jnp.zeros((_L,), jnp.float32)

    def coords(t):
        plane = wid * _PPW + t // _CHUNKS
        b = plane // _C
        c = plane % _C
        i0 = (t % _CHUNKS) * _CH
        return b, c, i0

    def start_in(t, buf):
        b, c, i0 = coords(t)
        pltpu.async_copy(x_hbm.at[b, c, pl.ds(i0, _CH), :], xv[buf], sin[buf])
        pltpu.async_copy(idx_hbm.at[b, c, pl.ds(i0, _CH), :], iv[buf], sin[buf])

    def wait_in(buf):
        pltpu.make_async_copy(x_hbm.at[0, 0, pl.ds(0, _CH), :], xv[buf],
                              sin[buf]).wait()
        pltpu.make_async_copy(idx_hbm.at[0, 0, pl.ds(0, _CH), :], iv[buf],
                              sin[buf]).wait()

    def wait_out(buf):
        pltpu.make_async_copy(ov[buf], out_hbm.at[0, 0, pl.ds(0, _OCH), :],
                              sout[buf]).wait()

    @plsc.parallel_loop(0, _OCH, step=1, unroll=2)
    def _(r):
        for v in range(_W // _L):
            ov0[r, pl.ds(v * _L, _L)] = zeros16
            ov1[r, pl.ds(v * _L, _L)] = zeros16

    start_in(0, 0)

    def step_pair(i, carry):
        for buf in range(2):
            t = 2 * i + buf

            @pl.when(i >= 1)
            def _():
                wait_out(buf)

                @plsc.parallel_loop(0, _CH, step=1, unroll=2)
                def _(k):
                    row0 = 2 * k
                    for v in range(_VPR):
                        t1 = rv[buf][k, pl.ds(v * _L, _L)]
                        di = t1 >= _W
                        col = jnp.where(di, t1 - _W, t1)
                        row = jnp.where(di, row0 + 1, row0)
                        plsc.store_scatter(ov[buf], [row, col], zeros16)

            wait_in(buf)

            if buf == 0:
                start_in(t + 1, 1)
            else:
                @pl.when(i < _T // 2 - 1)
                def _():
                    start_in(t + 1, 0)

            _, _, i0 = coords(t)

            @plsc.parallel_loop(0, _CH, step=1, unroll=2)
            def _(k):
                base = 768 * (i0 + k)
                row0 = 2 * k
                for v in range(_VPR):
                    xvec = xv[buf][k, pl.ds(v * _L, _L)]
                    ivec = iv[buf][k, pl.ds(v * _L, _L)]
                    t1 = ivec - base
                    rv[buf][k, pl.ds(v * _L, _L)] = t1
                    di = t1 >= _W
                    col = jnp.where(di, t1 - _W, t1)
                    row = jnp.where(di, row0 + 1, row0)
                    plsc.store_scatter(ov[buf], [row, col], xvec)

            b, c, i0 = coords(t)
            pltpu.async_copy(ov[buf], out_hbm.at[b, c, pl.ds(2 * i0, _OCH), :],
                             sout[buf])
        return carry

    lax.fori_loop(0, _T // 2, step_pair, 0)

    wait_out(0)
    wait_out(1)


@functools.partial(jax.jit)
def _unpool(x, indices):
    mesh = plsc.VectorSubcoreMesh(core_axis_name="c", subcore_axis_name="s")
    run = functools.partial(
        pl.kernel,
        mesh=mesh,
        out_type=jax.ShapeDtypeStruct((_B, _C, _H, _W), jnp.float32),
        compiler_params=pltpu.CompilerParams(needs_layout_passes=False),
        scratch_types=[
            pltpu.VMEM((_CH, _Wp), jnp.float32),
            pltpu.VMEM((_CH, _Wp), jnp.float32),
            pltpu.VMEM((_CH, _Wp), jnp.int32),
            pltpu.VMEM((_CH, _Wp), jnp.int32),
            pltpu.VMEM((_CH, _Wp), jnp.int32),
            pltpu.VMEM((_CH, _Wp), jnp.int32),
            pltpu.VMEM((_OCH, _W), jnp.float32),
            pltpu.VMEM((_OCH, _W), jnp.float32),
            pltpu.SemaphoreType.DMA,
            pltpu.SemaphoreType.DMA,
            pltpu.SemaphoreType.DMA,
            pltpu.SemaphoreType.DMA,
        ],
    )(_unpool_body)
    return run(x, indices)


def kernel(x, indices):
    return _unpool(x, indices)
